# baseline (speedup 1.0000x reference)
import jax
import jax.numpy as jnp
from jax import lax
from jax.experimental import pallas as pl
from jax.experimental.pallas import tpu as pltpu
from jax.experimental.pallas import tpu_sc as plsc

VOCAB = 1000000
D = 64
W = 200
B = 4096

NC = 2
NS = 16
NW = NC * NS

ROWS = B * W
ROWS_PER_W = ROWS // NW
CHUNKS_PER_W = ROWS_PER_W // W

NCOLS = VOCAB // 128
TAIL0 = NCOLS * 128
COLS_BASE = NCOLS // NW
COLS_REM = NCOLS % NW
K1_ITERS = COLS_BASE + 1


def _k1_body(tT_hbm, tail_hbm, tp_hbm, src_v, dst_v, isem, osem):
    wid = lax.axis_index("s") * NC + lax.axis_index("c")
    nfull = COLS_BASE + (wid < COLS_REM).astype(jnp.int32)
    base_col = wid * COLS_BASE + jnp.minimum(wid, COLS_REM)
    iota16 = lax.iota(jnp.int32, 16)

    def col_of(i):
        return base_col + jnp.where(i < nfull, i, 0)

    def in_copy(i, b):
        return pltpu.make_async_copy(
            tT_hbm.at[:, pl.ds(col_of(i) * 128, 128)], src_v.at[b],
            isem.at[b])

    def out_copy(i, b):
        return pltpu.make_async_copy(
            dst_v.at[b], tp_hbm.at[pl.ds(col_of(i) * 64, 64)], osem.at[b])

    in_copy(0, 0).start()

    @pl.loop(0, K1_ITERS + 1, step=2)
    def _blk(i0):
        for b in range(2):
            g = i0 + b

            @pl.when(g < K1_ITERS)
            def _one():
                in_copy(g, b).wait()

                @pl.when(g + 1 < K1_ITERS)
                def _pre():
                    in_copy(g + 1, 1 - b).start()

                @pl.when(g >= 2)
                def _drain():
                    out_copy(g - 2, b).wait()

                @pl.loop(0, 64)
                def _jj(jj):
                    for t in range(2):
                        vv = iota16 * 0 + (2 * jj + t)
                        for c in range(D // 16):
                            val = plsc.load_gather(
                                src_v.at[b], [iota16 + c * 16, vv])
                            dst_v[b, jj, pl.ds(t * D + c * 16, 16)] = val

                out_copy(g, b).start()

    @pl.when(K1_ITERS >= 2)
    def _d0():
        out_copy(K1_ITERS - 2, (K1_ITERS - 2) % 2).wait()

    out_copy(K1_ITERS - 1, (K1_ITERS - 1) % 2).wait()

    @pl.when(wid == NW - 1)
    def _tail():
        pltpu.sync_copy(tail_hbm, src_v.at[0, pl.ds(0, 32)])
        pltpu.sync_copy(src_v.at[0, pl.ds(0, 32)],
                        tp_hbm.at[pl.ds(TAIL0 // 2, 32)])


WP = 256


def _gather(tp_hbm, idx2_v, idxT_v, b, g_v, sem):
    c0 = pltpu.async_copy(
        tp_hbm.at[idx2_v.at[b, pl.ds(0, 128)]],
        g_v.at[b, pl.ds(0, 128)], sem)
    c1 = pltpu.async_copy(
        tp_hbm.at[idxT_v.at[b]],
        g_v.at[b, pl.ds(W - 128, 128)], sem)
    return c0, c1


def _k2_body(idx2_hbm, idxT_hbm, idxr_hbm, tp_hbm, pe_hbm, out_hbm,
             idx2_v, idxT_v, idxr_v, g_v, rows_v, pe_v, gsem, osem):
    wid = lax.axis_index("s") * NC + lax.axis_index("c")
    base_w = wid * ROWS_PER_W

    pltpu.sync_copy(pe_hbm, pe_v)

    def out_copy(b, base):
        return pltpu.make_async_copy(
            rows_v.at[b, pl.ds(0, W)], out_hbm.at[pl.ds(base, W)],
            osem.at[b])

    pltpu.sync_copy(idx2_hbm.at[wid, 0], idx2_v.at[0])
    pltpu.sync_copy(idxT_hbm.at[wid, 0], idxT_v.at[0])
    pltpu.sync_copy(idxr_hbm.at[wid, 0], idxr_v.at[0])
    _gather(tp_hbm, idx2_v, idxT_v, 0, g_v, gsem.at[0])

    @pl.loop(0, CHUNKS_PER_W, step=2)
    def _pair(g0):
        for b in range(2):
            g = g0 + b
            nb = 1 - b
            base = base_w + g * W

            @pl.when(g + 1 < CHUNKS_PER_W)
            def _prefetch():
                @pl.when(g >= 1)
                def _drain_prev_store():
                    out_copy(nb, base - W).wait()

                pltpu.sync_copy(idx2_hbm.at[wid, g + 1], idx2_v.at[nb])
                pltpu.sync_copy(idxT_hbm.at[wid, g + 1], idxT_v.at[nb])
                pltpu.sync_copy(idxr_hbm.at[wid, g + 1], idxr_v.at[nb])
                _gather(tp_hbm, idx2_v, idxT_v, nb, g_v, gsem.at[nb])

            c0 = pltpu.make_async_copy(
                tp_hbm.at[idx2_v.at[b, pl.ds(0, 128)]],
                g_v.at[b, pl.ds(0, 128)], gsem.at[b])
            c1 = pltpu.make_async_copy(
                tp_hbm.at[idxT_v.at[b]],
                g_v.at[b, pl.ds(W - 128, 128)], gsem.at[b])
            c0.wait()
            c1.wait()

            @pl.loop(0, (W + 15) // 16)
            def _grp(jg):
                w0 = pl.multiple_of(jg * 16, 16)
                vo = idxr_v[b, pl.ds(w0, 16)]
                for l in range(16):
                    w = w0 + l
                    off = pl.multiple_of((vo[l] & 1) << 6, 16)
                    for c in range(D // 16):
                        rows_v[b, w, pl.ds(c * 16, 16)] = (
                            g_v[b, w, pl.ds(off + c * 16, 16)]
                            + pe_v[jg * 8 + l // 2,
                                   pl.ds((l % 2) * D + c * 16, 16)])

            out_copy(b, base).start()

    out_copy((CHUNKS_PER_W - 1) % 2,
             base_w + (CHUNKS_PER_W - 1) * W).wait()


def kernel(x, table, pos_enc):
    idx = x.reshape(ROWS).astype(jnp.int32)
    flat2 = (idx >> 1).reshape(NW * CHUNKS_PER_W, W)
    idx2b = flat2[:, :128].reshape(NW, CHUNKS_PER_W, 128)
    idxTb = flat2[:, W - 128:].reshape(NW, CHUNKS_PER_W, 128)
    idxrb = jnp.pad(idx.reshape(NW * CHUNKS_PER_W, W),
                    ((0, 0), (0, WP - W))).reshape(NW, CHUNKS_PER_W, WP)
    tT = table.T
    tail2 = table[TAIL0:].reshape(D // 2, 2 * D)
    pe2 = jnp.pad(pos_enc.reshape(W // 2, 2 * D), ((0, 4), (0, 0)))
    mesh = plsc.VectorSubcoreMesh(core_axis_name="c", subcore_axis_name="s")

    t_pad = pl.kernel(
        _k1_body,
        out_type=jax.ShapeDtypeStruct((VOCAB // 2, 2 * D), jnp.float32),
        mesh=mesh,
        compiler_params=pltpu.CompilerParams(
            use_tc_tiling_on_sc=True, needs_layout_passes=False),
        scratch_types=[
            pltpu.VMEM((2, D, 128), jnp.float32),
            pltpu.VMEM((2, 64, 2 * D), jnp.float32),
            pltpu.SemaphoreType.DMA((2,)),
            pltpu.SemaphoreType.DMA((2,)),
        ],
    )(tT, tail2)

    out = pl.kernel(
        _k2_body,
        out_type=jax.ShapeDtypeStruct((ROWS, D), jnp.float32),
        mesh=mesh,
        compiler_params=pltpu.CompilerParams(use_tc_tiling_on_sc=True),
        scratch_types=[
            pltpu.VMEM((2, 128), jnp.int32),
            pltpu.VMEM((2, 128), jnp.int32),
            pltpu.VMEM((2, WP), jnp.int32),
            pltpu.VMEM((2, 208, 2 * D), jnp.float32),
            pltpu.VMEM((2, 208, D), jnp.float32),
            pltpu.VMEM((W // 2 + 4, 2 * D), jnp.float32),
            pltpu.SemaphoreType.DMA((2,)),
            pltpu.SemaphoreType.DMA((2,)),
        ],
    )(idx2b, idxTb, idxrb, t_pad, pe2)
    return out.reshape(B, W, D)


# speedup vs baseline: 1.2175x; 1.2175x over previous
import jax
import jax.numpy as jnp
from jax import lax
from jax.experimental import pallas as pl
from jax.experimental.pallas import tpu as pltpu
from jax.experimental.pallas import tpu_sc as plsc

VOCAB = 1000000
D = 64
W = 200
B = 4096

NC = 2
NS = 16
NW = NC * NS

ROWS = B * W
ROWS_PER_W = ROWS // NW
CHUNKS_PER_W = ROWS_PER_W // W

NCOLS = VOCAB // 128
TAIL0 = NCOLS * 128
COLS_BASE = NCOLS // NW
COLS_REM = NCOLS % NW
K1_ITERS = COLS_BASE + 1


def _k1_body(tT_hbm, tail_hbm, tp_hbm, src_v, dst_v, isem, osem):
    wid = lax.axis_index("s") * NC + lax.axis_index("c")
    nfull = COLS_BASE + (wid < COLS_REM).astype(jnp.int32)
    base_col = wid * COLS_BASE + jnp.minimum(wid, COLS_REM)
    pat = lax.iota(jnp.int32, 16)

    def col_of(i):
        return base_col + jnp.where(i < nfull, i, 0)

    def in_copy(i, b):
        return pltpu.make_async_copy(
            tT_hbm.at[:, pl.ds(col_of(i) * 128, 128)], src_v.at[b],
            isem.at[b])

    def out_copy(i, b):
        return pltpu.make_async_copy(
            dst_v.at[b], tp_hbm.at[pl.ds(col_of(i) * 128, 128)],
            osem.at[b])

    in_copy(0, 0).start()

    @pl.loop(0, K1_ITERS + 1, step=2)
    def _blk(i0):
        for b in range(2):
            g = i0 + b

            @pl.when(g < K1_ITERS)
            def _one():
                in_copy(g, b).wait()

                @pl.when(g + 1 < K1_ITERS)
                def _pre():
                    in_copy(g + 1, 1 - b).start()

                @pl.when(g >= 2)
                def _drain():
                    out_copy(g - 2, b).wait()

                @pl.loop(0, D)
                def _d(d):
                    dcol = jnp.full((16,), d, jnp.int32)
                    for c in range(8):
                        val = src_v[b, d, pl.ds(c * 16, 16)]
                        plsc.store_scatter(
                            dst_v.at[b], [pat + c * 16, dcol], val)

                out_copy(g, b).start()

    @pl.when(K1_ITERS >= 2)
    def _d0():
        out_copy(K1_ITERS - 2, (K1_ITERS - 2) % 2).wait()

    out_copy(K1_ITERS - 1, (K1_ITERS - 1) % 2).wait()

    @pl.when(wid == NW - 1)
    def _tail():
        pltpu.sync_copy(tail_hbm, src_v.at[0, :, :])
        pltpu.sync_copy(src_v.at[0, :, :], tp_hbm.at[pl.ds(TAIL0, 64)])


def _gather(tp_hbm, idx_v, idxT_v, b, g_v, sem):
    c0 = pltpu.async_copy(
        tp_hbm.at[idx_v.at[b]], g_v.at[b, pl.ds(0, 128)], sem)
    c1 = pltpu.async_copy(
        tp_hbm.at[idxT_v.at[b]], g_v.at[b, pl.ds(W - 128, 128)], sem)
    return c0, c1


def _k2_body(idx_hbm, idxT_hbm, tp_hbm, pe_hbm, out_hbm,
             idx_v, idxT_v, g_v, rows_v, pe_v, gsem, osem):
    wid = lax.axis_index("s") * NC + lax.axis_index("c")
    base_w = wid * ROWS_PER_W

    pltpu.sync_copy(pe_hbm, pe_v)

    def out_copy(b, base):
        return pltpu.make_async_copy(
            rows_v.at[b], out_hbm.at[pl.ds(base, W)], osem.at[b])

    pltpu.sync_copy(idx_hbm.at[wid, 0], idx_v.at[0])
    pltpu.sync_copy(idxT_hbm.at[wid, 0], idxT_v.at[0])
    _gather(tp_hbm, idx_v, idxT_v, 0, g_v, gsem.at[0])

    @pl.loop(0, CHUNKS_PER_W, step=2)
    def _pair(g0):
        for b in range(2):
            g = g0 + b
            nb = 1 - b
            base = base_w + g * W

            @pl.when(g + 1 < CHUNKS_PER_W)
            def _prefetch():
                @pl.when(g >= 1)
                def _drain_prev_store():
                    out_copy(nb, base - W).wait()

                pltpu.sync_copy(idx_hbm.at[wid, g + 1], idx_v.at[nb])
                pltpu.sync_copy(idxT_hbm.at[wid, g + 1], idxT_v.at[nb])
                _gather(tp_hbm, idx_v, idxT_v, nb, g_v, gsem.at[nb])

            c0 = pltpu.make_async_copy(
                tp_hbm.at[idx_v.at[b]], g_v.at[b, pl.ds(0, 128)],
                gsem.at[b])
            c1 = pltpu.make_async_copy(
                tp_hbm.at[idxT_v.at[b]], g_v.at[b, pl.ds(W - 128, 128)],
                gsem.at[b])
            c0.wait()
            c1.wait()

            @pl.loop(0, W // 2)
            def _row(j):
                for t in range(2):
                    w = 2 * j + t
                    for c in range(D // 16):
                        rows_v[b, w, pl.ds(c * 16, 16)] = (
                            g_v[b, w, pl.ds(c * 16, 16)]
                            + pe_v[j, pl.ds(t * D + c * 16, 16)])

            out_copy(b, base).start()

    out_copy((CHUNKS_PER_W - 1) % 2,
             base_w + (CHUNKS_PER_W - 1) * W).wait()


def kernel(x, table, pos_enc):
    idx = x.reshape(ROWS).astype(jnp.int32)
    flat = idx.reshape(NW * CHUNKS_PER_W, W)
    idxb = flat[:, :128].reshape(NW, CHUNKS_PER_W, 128)
    idxTb = flat[:, W - 128:].reshape(NW, CHUNKS_PER_W, 128)
    tT = table.T
    tail2 = jnp.pad(table[TAIL0:], ((0, 0), (0, D)))
    pe2 = pos_enc.reshape(W // 2, 2 * D)
    mesh = plsc.VectorSubcoreMesh(core_axis_name="c", subcore_axis_name="s")

    t_pad = pl.kernel(
        _k1_body,
        out_type=jax.ShapeDtypeStruct((VOCAB, 2 * D), jnp.float32),
        mesh=mesh,
        compiler_params=pltpu.CompilerParams(
            use_tc_tiling_on_sc=True, needs_layout_passes=False),
        scratch_types=[
            pltpu.VMEM((2, D, 128), jnp.float32),
            pltpu.VMEM((2, 128, 2 * D), jnp.float32),
            pltpu.SemaphoreType.DMA((2,)),
            pltpu.SemaphoreType.DMA((2,)),
        ],
    )(tT, tail2)

    out = pl.kernel(
        _k2_body,
        out_type=jax.ShapeDtypeStruct((ROWS, D), jnp.float32),
        mesh=mesh,
        compiler_params=pltpu.CompilerParams(use_tc_tiling_on_sc=True),
        scratch_types=[
            pltpu.VMEM((2, 128), jnp.int32),
            pltpu.VMEM((2, 128), jnp.int32),
            pltpu.VMEM((2, W, 2 * D), jnp.float32),
            pltpu.VMEM((2, W, D), jnp.float32),
            pltpu.VMEM((W // 2, 2 * D), jnp.float32),
            pltpu.SemaphoreType.DMA((2,)),
            pltpu.SemaphoreType.DMA((2,)),
        ],
    )(idxb, idxTb, t_pad, pe2)
    return out.reshape(B, W, D)


# speedup vs baseline: 1.8108x; 1.4873x over previous
import jax
import jax.numpy as jnp
from jax import lax
from jax.experimental import pallas as pl
from jax.experimental.pallas import tpu as pltpu
from jax.experimental.pallas import tpu_sc as plsc

VOCAB = 1000000
D = 64
W = 200
B = 4096

NC = 2
NS = 16
NW = NC * NS

ROWS = B * W
ROWS_PER_W = ROWS // NW
CHUNKS_PER_W = ROWS_PER_W // W

NCOLS = VOCAB // 128
TAIL0 = NCOLS * 128
COLS_BASE = NCOLS // NW
COLS_REM = NCOLS % NW
K1_ITERS = COLS_BASE + 1


def _k1_body(tT_hbm, tail_hbm, tp_hbm, src_v, dst_v, isem, osem):
    wid = lax.axis_index("s") * NC + lax.axis_index("c")
    nfull = COLS_BASE + (wid < COLS_REM).astype(jnp.int32)
    base_col = wid * COLS_BASE + jnp.minimum(wid, COLS_REM)
    pat = lax.iota(jnp.int32, 16)

    def col_of(i):
        return base_col + jnp.where(i < nfull, i, 0)

    def in_copy(i, b):
        return pltpu.make_async_copy(
            tT_hbm.at[:, pl.ds(col_of(i) * 128, 128)], src_v.at[b],
            isem.at[b])

    def out_copy(i, b):
        return pltpu.make_async_copy(
            dst_v.at[b], tp_hbm.at[pl.ds(col_of(i) * 128, 128)],
            osem.at[b])

    in_copy(0, 0).start()

    @pl.loop(0, K1_ITERS + 1, step=2)
    def _blk(i0):
        for b in range(2):
            g = i0 + b

            @pl.when(g < K1_ITERS)
            def _one():
                in_copy(g, b).wait()

                @pl.when(g + 1 < K1_ITERS)
                def _pre():
                    in_copy(g + 1, 1 - b).start()

                @pl.when(g >= 2)
                def _drain():
                    out_copy(g - 2, b).wait()

                @pl.loop(0, D)
                def _d(d):
                    dcol = jnp.full((16,), d, jnp.int32)
                    for c in range(8):
                        val = src_v[b, d, pl.ds(c * 16, 16)]
                        plsc.store_scatter(
                            dst_v.at[b], [pat + c * 16, dcol], val)

                out_copy(g, b).start()

    @pl.when(K1_ITERS >= 2)
    def _d0():
        out_copy(K1_ITERS - 2, (K1_ITERS - 2) % 2).wait()

    out_copy(K1_ITERS - 1, (K1_ITERS - 1) % 2).wait()

    @pl.when(wid == NW - 1)
    def _tail():
        pltpu.sync_copy(tail_hbm, src_v.at[0, :, :])
        pltpu.sync_copy(src_v.at[0, :, :], tp_hbm.at[pl.ds(TAIL0, 64)])


def _gather(tp_hbm, idx_v, idxT_v, b, g_v, sem):
    c0 = pltpu.async_copy(
        tp_hbm.at[idx_v.at[b]], g_v.at[b, pl.ds(0, 128)], sem)
    c1 = pltpu.async_copy(
        tp_hbm.at[idxT_v.at[b]], g_v.at[b, pl.ds(W - 128, 128)], sem)
    return c0, c1


def _k2_body(idx_hbm, idxT_hbm, tp_hbm, pe_hbm, out_hbm,
             idx_v, idxT_v, g_v, rows_v, pe_v, gsem, osem):
    wid = lax.axis_index("s") * NC + lax.axis_index("c")
    base_w = wid * ROWS_PER_W

    pltpu.sync_copy(pe_hbm, pe_v)

    def out_copy(b, base):
        return pltpu.make_async_copy(
            rows_v.at[b], out_hbm.at[pl.ds(base, W)], osem.at[b])

    pltpu.sync_copy(idx_hbm.at[wid, 0], idx_v.at[0])
    pltpu.sync_copy(idxT_hbm.at[wid, 0], idxT_v.at[0])
    _gather(tp_hbm, idx_v, idxT_v, 0, g_v, gsem.at[0])

    @pl.loop(0, CHUNKS_PER_W, step=2)
    def _pair(g0):
        for b in range(2):
            g = g0 + b
            nb = 1 - b
            base = base_w + g * W

            @pl.when(g + 1 < CHUNKS_PER_W)
            def _prefetch():
                @pl.when(g >= 1)
                def _drain_prev_store():
                    out_copy(nb, base - W).wait()

                pltpu.sync_copy(idx_hbm.at[wid, g + 1], idx_v.at[nb])
                pltpu.sync_copy(idxT_hbm.at[wid, g + 1], idxT_v.at[nb])
                _gather(tp_hbm, idx_v, idxT_v, nb, g_v, gsem.at[nb])

            c0 = pltpu.make_async_copy(
                tp_hbm.at[idx_v.at[b]], g_v.at[b, pl.ds(0, 128)],
                gsem.at[b])
            c1 = pltpu.make_async_copy(
                tp_hbm.at[idxT_v.at[b]], g_v.at[b, pl.ds(W - 128, 128)],
                gsem.at[b])
            c0.wait()
            c1.wait()

            @pl.loop(0, W // 2)
            def _row(j):
                for t in range(2):
                    w = 2 * j + t
                    for c in range(D // 16):
                        rows_v[b, w, pl.ds(c * 16, 16)] = (
                            g_v[b, w, pl.ds(c * 16, 16)]
                            + pe_v[j, pl.ds(t * D + c * 16, 16)])

            out_copy(b, base).start()

    out_copy((CHUNKS_PER_W - 1) % 2,
             base_w + (CHUNKS_PER_W - 1) * W).wait()


def kernel(x, table, pos_enc):
    idx = x.reshape(ROWS).astype(jnp.int32)
    flat = idx.reshape(NW * CHUNKS_PER_W, W)
    idxb = flat[:, :128].reshape(NW, CHUNKS_PER_W, 128)
    idxTb = flat[:, W - 128:].reshape(NW, CHUNKS_PER_W, 128)
    pe2 = pos_enc.reshape(W // 2, 2 * D)
    mesh = plsc.VectorSubcoreMesh(core_axis_name="c", subcore_axis_name="s")

    t_pad = jnp.pad(table, ((0, 0), (0, D)))

    out = pl.kernel(
        _k2_body,
        out_type=jax.ShapeDtypeStruct((ROWS, D), jnp.float32),
        mesh=mesh,
        compiler_params=pltpu.CompilerParams(use_tc_tiling_on_sc=True),
        scratch_types=[
            pltpu.VMEM((2, 128), jnp.int32),
            pltpu.VMEM((2, 128), jnp.int32),
            pltpu.VMEM((2, W, 2 * D), jnp.float32),
            pltpu.VMEM((2, W, D), jnp.float32),
            pltpu.VMEM((W // 2, 2 * D), jnp.float32),
            pltpu.SemaphoreType.DMA((2,)),
            pltpu.SemaphoreType.DMA((2,)),
        ],
    )(idxb, idxTb, t_pad, pe2)
    return out.reshape(B, W, D)


# speedup vs baseline: 1.8120x; 1.0006x over previous
import jax
import jax.numpy as jnp
from jax import lax
from jax.experimental import pallas as pl
from jax.experimental.pallas import tpu as pltpu
from jax.experimental.pallas import tpu_sc as plsc

VOCAB = 1000000
D = 64
W = 200
B = 4096

NC = 2
NS = 16
NW = NC * NS

ROWS = B * W
ROWS_PER_W = ROWS // NW
CHUNKS_PER_W = ROWS_PER_W // W


def _gather(tp_hbm, idx_v, idxT_v, b, g_v, sem):
    c0 = pltpu.async_copy(
        tp_hbm.at[idx_v.at[b]], g_v.at[b, pl.ds(0, 128)], sem)
    c1 = pltpu.async_copy(
        tp_hbm.at[idxT_v.at[b]], g_v.at[b, pl.ds(W - 128, 128)], sem)
    return c0, c1


def _k2_body(idx_hbm, idxT_hbm, tp_hbm, pe_hbm, out_hbm,
             idx_v, idxT_v, g_v, rows_v, pe_v, gsem, osem):
    wid = lax.axis_index("s") * NC + lax.axis_index("c")
    base_w = wid * ROWS_PER_W

    pltpu.sync_copy(pe_hbm, pe_v)

    def out_copy(b, base):
        return pltpu.make_async_copy(
            rows_v.at[b], out_hbm.at[pl.ds(base, W)], osem.at[b])

    pltpu.sync_copy(idx_hbm.at[wid, 0], idx_v.at[0])
    pltpu.sync_copy(idxT_hbm.at[wid, 0], idxT_v.at[0])
    _gather(tp_hbm, idx_v, idxT_v, 0, g_v, gsem.at[0])

    @pl.loop(0, CHUNKS_PER_W, step=2)
    def _pair(g0):
        for b in range(2):
            g = g0 + b
            nb = 1 - b
            base = base_w + g * W

            @pl.when(g + 1 < CHUNKS_PER_W)
            def _prefetch():
                @pl.when(g >= 1)
                def _drain_prev_store():
                    out_copy(nb, base - W).wait()

                pltpu.sync_copy(idx_hbm.at[wid, g + 1], idx_v.at[nb])
                pltpu.sync_copy(idxT_hbm.at[wid, g + 1], idxT_v.at[nb])
                _gather(tp_hbm, idx_v, idxT_v, nb, g_v, gsem.at[nb])

            c0 = pltpu.make_async_copy(
                tp_hbm.at[idx_v.at[b]], g_v.at[b, pl.ds(0, 128)],
                gsem.at[b])
            c1 = pltpu.make_async_copy(
                tp_hbm.at[idxT_v.at[b]], g_v.at[b, pl.ds(W - 128, 128)],
                gsem.at[b])
            c0.wait()
            c1.wait()

            @pl.loop(0, W // 2)
            def _row(j):
                for t in range(2):
                    w = 2 * j + t
                    for c in range(D // 16):
                        rows_v[b, w, pl.ds(c * 16, 16)] = (
                            g_v[b, w, pl.ds(c * 16, 16)]
                            + pe_v[j, pl.ds(t * D + c * 16, 16)])

            out_copy(b, base).start()

    out_copy((CHUNKS_PER_W - 1) % 2,
             base_w + (CHUNKS_PER_W - 1) * W).wait()


def kernel(x, table, pos_enc):
    idx = x.reshape(ROWS).astype(jnp.int32)
    flat = idx.reshape(NW * CHUNKS_PER_W, W)
    idxb = flat[:, :128].reshape(NW, CHUNKS_PER_W, 128)
    idxTb = flat[:, W - 128:].reshape(NW, CHUNKS_PER_W, 128)
    pe2 = pos_enc.reshape(W // 2, 2 * D)
    mesh = plsc.VectorSubcoreMesh(core_axis_name="c", subcore_axis_name="s")

    t_pad = jnp.pad(table, ((0, 0), (0, D)))

    out = pl.kernel(
        _k2_body,
        out_type=jax.ShapeDtypeStruct((ROWS, D), jnp.float32),
        mesh=mesh,
        compiler_params=pltpu.CompilerParams(use_tc_tiling_on_sc=True),
        scratch_types=[
            pltpu.VMEM((2, 128), jnp.int32),
            pltpu.VMEM((2, 128), jnp.int32),
            pltpu.VMEM((2, W, 2 * D), jnp.float32),
            pltpu.VMEM((2, W, D), jnp.float32),
            pltpu.VMEM((W // 2, 2 * D), jnp.float32),
            pltpu.SemaphoreType.DMA((2,)),
            pltpu.SemaphoreType.DMA((2,)),
        ],
    )(idxb, idxTb, t_pad, pe2)
    return out.reshape(B, W, D)
